# baseline (speedup 1.0000x reference)
import functools

import jax
import jax.numpy as jnp
from jax import lax
from jax.experimental import pallas as pl
from jax.experimental.pallas import tpu as pltpu
from jax.experimental.pallas import tpu_sc as plsc

_NC = 2
_NS = 16
_NW = _NC * _NS
_LANE = 128


def _iota16():
    return lax.iota(jnp.int32, 16)


def _splat16(v):
    return jnp.full((16,), v, jnp.int32)


@functools.lru_cache(maxsize=None)
def _make_gather(B, V, D):
    n_blocks = B // _LANE
    blocks_per_w = n_blocks // _NW
    n_bands = D // 8
    mesh = plsc.VectorSubcoreMesh(core_axis_name="c", subcore_axis_name="s")

    @functools.partial(
        pl.kernel,
        out_type=jax.ShapeDtypeStruct((n_bands, n_blocks, 8, _LANE), jnp.float32),
        mesh=mesh,
        scratch_types=[
            pltpu.VMEM((blocks_per_w, _LANE), jnp.int32),
            pltpu.VMEM((_LANE, D), jnp.float32),
            pltpu.VMEM((_LANE, D), jnp.float32),
            pltpu.VMEM((_LANE, D), jnp.float32),
            pltpu.VMEM((_LANE, D), jnp.float32),
            pltpu.VMEM((D, _LANE), jnp.float32),
            pltpu.VMEM((D, _LANE), jnp.float32),
            pltpu.SemaphoreType.DMA,
            pltpu.SemaphoreType.DMA,
            pltpu.SemaphoreType.DMA,
            pltpu.SemaphoreType.DMA,
            pltpu.SemaphoreType.DMA,
            pltpu.SemaphoreType.DMA,
        ],
        compiler_params=pltpu.CompilerParams(
            use_tc_tiling_on_sc=False, needs_layout_passes=False
        ),
    )
    def gather_kernel(idx_hbm, table_hbm, out4, idx_v,
                      rows_a, rows_b, rows_c, rows_d,
                      band_a, band_b,
                      gsem_a, gsem_b, gsem_c, gsem_d, osem_a, osem_b):
        rows = (rows_a, rows_b, rows_c, rows_d)
        bands = (band_a, band_b)
        gsems = (gsem_a, gsem_b, gsem_c, gsem_d)
        osems = (osem_a, osem_b)
        wid = lax.axis_index("s") * _NC + lax.axis_index("c")
        base = wid * blocks_per_w
        pltpu.sync_copy(idx_hbm.at[pl.ds(base, blocks_per_w)], idx_v)

        def fire(slot, i):
            pltpu.async_copy(table_hbm.at[idx_v.at[i]], rows[slot], gsems[slot])

        def wait_gather(slot):
            pltpu.make_async_copy(
                table_hbm.at[pl.ds(0, _LANE)], rows[slot], gsems[slot]
            ).wait()

        def write_bands(slot, nt):
            for g in range(n_bands):
                pltpu.async_copy(
                    bands[slot].at[pl.ds(8 * g, 8)], out4.at[g, nt], osems[slot]
                )

        def wait_bands(slot):
            for g in range(n_bands):
                pltpu.make_async_copy(
                    bands[slot].at[pl.ds(8 * g, 8)], out4.at[g, 0], osems[slot]
                ).wait()

        iotas = [16 * lg + _iota16() for lg in range(8)]

        def transpose(rslot, bslot):
            src, dst = rows[rslot], bands[bslot]

            @plsc.parallel_loop(0, D, unroll=4)
            def _(f):
                sf = _splat16(f)
                for lg in range(8):
                    v = plsc.load_gather(src, [iotas[lg], sf])
                    dst[f, pl.ds(16 * lg, 16)] = v

        for s in range(3):
            fire(s, s)

        def step(t4, carry):
            for b in range(4):
                i = 4 * t4 + b

                @pl.when(i + 3 < blocks_per_w)
                def _():
                    fire((b + 3) % 4, i + 3)

                wait_gather(b)

                @pl.when(i >= 2)
                def _():
                    wait_bands(b % 2)

                transpose(b, b % 2)
                write_bands(b % 2, base + i)
            return carry

        lax.fori_loop(0, blocks_per_w // 4, step, 0)
        wait_bands(0)
        wait_bands(1)

    return gather_kernel


def kernel(walks, table):
    (B,) = walks.shape
    V, D = table.shape
    idx2d = walks.astype(jnp.int32).reshape(B // _LANE, _LANE)
    out4 = _make_gather(B, V, D)(idx2d, table)
    return out4.transpose(1, 3, 0, 2).reshape(B, D)


# speedup vs baseline: 1.5325x; 1.5325x over previous
import functools

import jax
import jax.numpy as jnp
from jax import lax
from jax.experimental import pallas as pl
from jax.experimental.pallas import tpu as pltpu
from jax.experimental.pallas import tpu_sc as plsc

_NC = 2
_NS = 16
_NW = _NC * _NS
_LANE = 128


def _iota16():
    return lax.iota(jnp.int32, 16)


def _splat16(v):
    return jnp.full((16,), v, jnp.int32)


@functools.lru_cache(maxsize=None)
def _make_gather(B, V, D):
    n_blocks = B // _LANE
    blocks_per_w = n_blocks // _NW
    n_bands = D // 8
    mesh = plsc.VectorSubcoreMesh(core_axis_name="c", subcore_axis_name="s")

    @functools.partial(
        pl.kernel,
        out_type=jax.ShapeDtypeStruct((n_bands, n_blocks, 8, _LANE), jnp.float32),
        mesh=mesh,
        scratch_types=[
            pltpu.VMEM((blocks_per_w, _LANE), jnp.int32),
            pltpu.VMEM((_LANE, D), jnp.float32),
            pltpu.VMEM((_LANE, D), jnp.float32),
            pltpu.VMEM((_LANE, D), jnp.float32),
            pltpu.VMEM((_LANE, D), jnp.float32),
            pltpu.VMEM((D, _LANE + 1), jnp.float32),
            pltpu.VMEM((D, _LANE + 1), jnp.float32),
            pltpu.SemaphoreType.DMA,
            pltpu.SemaphoreType.DMA,
            pltpu.SemaphoreType.DMA,
            pltpu.SemaphoreType.DMA,
            pltpu.SemaphoreType.DMA,
            pltpu.SemaphoreType.DMA,
        ],
        compiler_params=pltpu.CompilerParams(
            use_tc_tiling_on_sc=False, needs_layout_passes=False
        ),
    )
    def gather_kernel(idx_hbm, table_hbm, out4, idx_v,
                      rows_a, rows_b, rows_c, rows_d,
                      band_a, band_b,
                      gsem_a, gsem_b, gsem_c, gsem_d, osem_a, osem_b):
        rows = (rows_a, rows_b, rows_c, rows_d)
        bands = (band_a, band_b)
        gsems = (gsem_a, gsem_b, gsem_c, gsem_d)
        osems = (osem_a, osem_b)
        wid = lax.axis_index("s") * _NC + lax.axis_index("c")
        base = wid * blocks_per_w
        pltpu.sync_copy(idx_hbm.at[pl.ds(base, blocks_per_w)], idx_v)

        def fire(slot, i):
            pltpu.async_copy(table_hbm.at[idx_v.at[i]], rows[slot], gsems[slot])

        def wait_gather(slot):
            pltpu.make_async_copy(
                table_hbm.at[pl.ds(0, _LANE)], rows[slot], gsems[slot]
            ).wait()

        def write_bands(slot, nt):
            for g in range(n_bands):
                pltpu.async_copy(
                    bands[slot].at[pl.ds(8 * g, 8), pl.ds(0, _LANE)],
                    out4.at[g, nt],
                    osems[slot],
                )

        def wait_bands(slot):
            for g in range(n_bands):
                pltpu.make_async_copy(
                    bands[slot].at[pl.ds(8 * g, 8), pl.ds(0, _LANE)],
                    out4.at[g, 0],
                    osems[slot],
                ).wait()

        iotas = [16 * h + _iota16() for h in range(D // 16)]

        def transpose(rslot, bslot):
            src, dst = rows[rslot], bands[bslot]

            @plsc.parallel_loop(0, _LANE, unroll=4)
            def _(l):
                sl = _splat16(l)
                for h in range(D // 16):
                    v = src[l, pl.ds(16 * h, 16)]
                    plsc.store_scatter(dst, [iotas[h], sl], v)

        for s in range(3):
            fire(s, s)

        def step(t4, carry):
            for b in range(4):
                i = 4 * t4 + b

                @pl.when(i + 3 < blocks_per_w)
                def _():
                    fire((b + 3) % 4, i + 3)

                wait_gather(b)

                @pl.when(i >= 2)
                def _():
                    wait_bands(b % 2)

                transpose(b, b % 2)
                write_bands(b % 2, base + i)
            return carry

        lax.fori_loop(0, blocks_per_w // 4, step, 0)
        wait_bands(0)
        wait_bands(1)

    return gather_kernel


def kernel(walks, table):
    (B,) = walks.shape
    V, D = table.shape
    idx2d = walks.astype(jnp.int32).reshape(B // _LANE, _LANE)
    out4 = _make_gather(B, V, D)(idx2d, table)
    return out4.transpose(1, 3, 0, 2).reshape(B, D)
